# baseline (speedup 1.0000x reference)
import functools

import jax
import jax.numpy as jnp
from jax.experimental import pallas as pl
from jax.experimental.pallas import tpu as pltpu

_N = 10000
_G = 64
_H = 256
_ROWS_BLK = 1000


def _mlp_body(z_ref, w1_ref, b1_ref, w2_ref, b2_ref, sc_ref, sh_ref, out_ref):
    z = z_ref[...]
    a = jnp.maximum(
        jax.lax.dot_general(z, w1_ref[...], (((1,), (0,)), ((), ())),
                            preferred_element_type=jnp.float32) + b1_ref[...], 0.0)
    b = jnp.maximum(
        jax.lax.dot_general(a, w2_ref[...], (((1,), (0,)), ((), ())),
                            preferred_element_type=jnp.float32) + b2_ref[...], 0.0)
    out_ref[...] = b * sc_ref[...] + sh_ref[...]


def _mlp_bn(z, w1, b1, w2, b2, scale, shift):
    n = z.shape[0]
    grid = n // _ROWS_BLK
    full = lambda s: pl.BlockSpec(s, lambda i: (0,) * len(s))
    return pl.pallas_call(
        _mlp_body,
        grid=(grid,),
        in_specs=[
            pl.BlockSpec((_ROWS_BLK, _H), lambda i: (i, 0)),
            full((_H, _H)), full((_H,)), full((_H, _H)), full((_H,)),
            full((_H,)), full((_H,)),
        ],
        out_specs=pl.BlockSpec((_ROWS_BLK, _H), lambda i: (i, 0)),
        out_shape=jax.ShapeDtypeStruct((n, _H), jnp.float32),
    )(z, w1, b1, w2, b2, scale, shift)


def _head_body(p_ref, w1_ref, b1_ref, w2_ref, b2_ref, out_ref):
    o = jnp.maximum(
        jax.lax.dot_general(p_ref[...], w1_ref[...], (((1,), (0,)), ((), ())),
                            preferred_element_type=jnp.float32) + b1_ref[...], 0.0)
    o = jax.lax.dot_general(o, w2_ref[...], (((1,), (0,)), ((), ())),
                            preferred_element_type=jnp.float32) + b2_ref[...]
    m = jnp.max(o, axis=-1, keepdims=True)
    lse = jnp.log(jnp.sum(jnp.exp(o - m), axis=-1, keepdims=True)) + m
    out_ref[...] = o - lse


def _head(pooled, w1, b1, w2, b2):
    c = w2.shape[1]
    return pl.pallas_call(
        _head_body,
        out_shape=jax.ShapeDtypeStruct((pooled.shape[0], c), jnp.float32),
    )(pooled, w1, b1, w2, b2)


def kernel(x, edge_index, batch, params):
    src = edge_index[0]
    dst = edge_index[1]
    h = x
    for layer in params['convs']:
        msgs = jnp.take(h, src, axis=0)
        agg = jax.ops.segment_sum(msgs, dst, num_segments=_N)
        z = agg + h
        scale = layer['bn_gamma'] * jax.lax.rsqrt(layer['bn_var'] + 1e-5)
        shift = layer['bn_beta'] - layer['bn_mean'] * scale
        h = _mlp_bn(z, layer['W1'], layer['b1'], layer['W2'], layer['b2'],
                    scale, shift)
    pooled = jax.ops.segment_max(h, batch, num_segments=_G)
    return _head(pooled, params['fc1_W'], params['fc1_b'],
                 params['fc2_W'], params['fc2_b'])
